# baseline (speedup 1.0000x reference)
import functools

import jax
import jax.numpy as jnp
import numpy as np
from jax import lax
from jax.experimental import pallas as pl
from jax.experimental.pallas import tpu as pltpu
from jax.experimental.pallas import tpu_sc as plsc

_N, _L, _D = 64, 1024, 768
_MASKING_RATIO = 0.75
_LEN_KEEP = int(_L * (1 - _MASKING_RATIO))
_B = _N * _LEN_KEEP
_NW = 32
_BPW = _B // _NW
_CHUNK = 32
_NCH = _BPW // _CHUNK

_cache = {}


def _rotl32(x, d):
    return ((x << np.uint32(d)) | (x >> np.uint32(32 - d))).astype(np.uint32)


def _threefry2x32(k0, k1, x0, x1):
    ks = [np.uint32(k0), np.uint32(k1),
          np.uint32(np.uint32(k0) ^ np.uint32(k1) ^ np.uint32(0x1BD11BDA))]
    rot_a = (13, 15, 26, 6)
    rot_b = (17, 29, 16, 24)
    x0 = (x0 + ks[0]).astype(np.uint32)
    x1 = (x1 + ks[1]).astype(np.uint32)
    for i, rots in enumerate((rot_a, rot_b, rot_a, rot_b, rot_a)):
        for r in rots:
            x0 = (x0 + x1).astype(np.uint32)
            x1 = _rotl32(x1, r)
            x1 = (x1 ^ x0).astype(np.uint32)
        x0 = (x0 + ks[(i + 1) % 3]).astype(np.uint32)
        x1 = (x1 + ks[(i + 2) % 3] + np.uint32(i + 1)).astype(np.uint32)
    return x0, x1


def _noise_constant():
    n = _N * _L
    hi = np.zeros(n, dtype=np.uint32)
    lo = np.arange(n, dtype=np.uint32)
    o0, o1 = _threefry2x32(0, 1, hi, lo)
    bits = o0 ^ o1
    f = ((bits >> np.uint32(9)) | np.uint32(0x3F800000)).view(np.float32)
    return np.maximum(np.float32(0.0), f - np.float32(1.0)).reshape(_N, _L)


def _consts():
    if not _cache:
        noise = _noise_constant()
        ids_shuffle = np.argsort(noise, axis=1, kind="stable").astype(np.int32)
        ids_restore = np.argsort(ids_shuffle, axis=1, kind="stable").astype(np.int32)
        ids_keep = ids_shuffle[:, :_LEN_KEEP]
        mask = (ids_restore >= _LEN_KEEP).astype(np.float32)
        g_idx = (
            ids_keep.astype(np.int64)
            + np.arange(_N, dtype=np.int64)[:, None] * _L
        ).reshape(-1).astype(np.int32)
        _cache.update(ids_restore=ids_restore, mask=mask, g_idx=g_idx)
    return _cache


_NBUF = 4
_NGRP = _NCH // _NBUF


def _make_gather():
    mesh = plsc.VectorSubcoreMesh(core_axis_name="c", subcore_axis_name="s")

    @functools.partial(
        pl.kernel,
        mesh=mesh,
        out_type=jax.ShapeDtypeStruct((_B, _D), jnp.float32),
        scratch_types=(
            [pltpu.VMEM((_BPW,), jnp.int32)]
            + [pltpu.VMEM((_CHUNK, _D), jnp.float32) for _ in range(_NBUF)]
            + [pltpu.SemaphoreType.DMA for _ in range(2 * _NBUF)]
        ),
    )
    def k(x_hbm, idx_hbm, out_hbm, idx_v, *bufs):
        rows = bufs[:_NBUF]
        gsem = bufs[_NBUF : 2 * _NBUF]
        osem = bufs[2 * _NBUF :]
        wid = lax.axis_index("s") * 2 + lax.axis_index("c")
        base = wid * _BPW
        pltpu.sync_copy(idx_hbm.at[pl.ds(base, _BPW)], idx_v)

        def gather(b, ci):
            off = pl.multiple_of(ci * _CHUNK, _CHUNK)
            return pltpu.make_async_copy(
                x_hbm.at[idx_v.at[pl.ds(off, _CHUNK)]], rows[b], gsem[b]
            )

        def put(b, ci):
            return pltpu.make_async_copy(
                rows[b], out_hbm.at[pl.ds(base + ci * _CHUNK, _CHUNK)], osem[b]
            )

        for b in range(_NBUF):
            gather(b, b).start()

        def body(g, carry):
            for b in range(_NBUF):
                ci = g * _NBUF + b
                gather(b, ci).wait()
                put(b, ci).start()

                @pl.when(g < _NGRP - 1)
                def _():
                    put(b, ci).wait()
                    gather(b, ci + _NBUF).start()

            return carry

        lax.fori_loop(0, _NGRP, body, 0)
        for b in range(_NBUF):
            put(b, (_NGRP - 1) * _NBUF + b).wait()

    return k


_gather = _make_gather()
_consts()


def kernel(x, img_pat):
    c = _consts()
    x_flat = x.reshape(_N * _L, _D)
    out = _gather(x_flat, jnp.asarray(c["g_idx"]))
    return (
        out.reshape(_N, _LEN_KEEP, _D),
        jnp.asarray(c["mask"]),
        jnp.asarray(c["ids_restore"]),
    )
